# baseline (speedup 1.0000x reference)
import functools

import jax
import jax.numpy as jnp
from jax import lax
from jax.experimental import pallas as pl
from jax.experimental.pallas import tpu as pltpu
from jax.experimental.pallas import tpu_sc as plsc

N = 10000
E = 320000
IN_C = 128
HID = 256
OUT_C = 64

NC = 2
NS = 16
NW = NC * NS
CH = 128
E_PAD = 327680
RPW = E_PAD // (NW * CH)
NP2 = 10240
ZPT = NP2 // NS

_MESH = plsc.VectorSubcoreMesh(core_axis_name="c", subcore_axis_name="s")


@functools.partial(
    pl.kernel,
    mesh=_MESH,
    out_type=jax.ShapeDtypeStruct((NC, NP2, IN_C), jnp.float32),
    scratch_types=[
        pltpu.VMEM((RPW, CH), jnp.int32),
        pltpu.VMEM((CH, IN_C), jnp.float32),
        pltpu.VMEM_SHARED((NP2, IN_C), jnp.float32),
    ],
)
def _deg_sc(dst_hbm, ones_hbm, z_hbm, out_hbm, idx_v, ones_v, accum):
    c = lax.axis_index("c")
    s = lax.axis_index("s")
    w = c * NS + s
    r0 = s * ZPT
    pltpu.sync_copy(z_hbm.at[pl.ds(r0, ZPT)], accum.at[pl.ds(r0, ZPT)])
    pltpu.sync_copy(dst_hbm.at[w], idx_v)
    pltpu.sync_copy(ones_hbm, ones_v)
    plsc.subcore_barrier()

    def step(k, _):
        pltpu.sync_copy(ones_v, accum.at[idx_v.at[k]], add=True)
        return 0

    lax.fori_loop(0, RPW, step, 0)
    plsc.subcore_barrier()
    pltpu.sync_copy(accum.at[pl.ds(r0, ZPT)], out_hbm.at[c, pl.ds(r0, ZPT)])


@functools.partial(
    pl.kernel,
    mesh=_MESH,
    out_type=jax.ShapeDtypeStruct((NC, NP2, IN_C), jnp.float32),
    scratch_types=[
        pltpu.VMEM((RPW, CH), jnp.int32),
        pltpu.VMEM((RPW, CH), jnp.int32),
        pltpu.VMEM((CH, IN_C), jnp.float32),
        pltpu.VMEM_SHARED((NP2, IN_C), jnp.float32),
        pltpu.SemaphoreType.DMA,
    ],
)
def _agg_sc(src_hbm, dst_hbm, g_hbm, z_hbm, out_hbm, si_v, di_v, rows_v, accum, sem):
    c = lax.axis_index("c")
    s = lax.axis_index("s")
    w = c * NS + s
    r0 = s * ZPT
    pltpu.sync_copy(z_hbm.at[pl.ds(r0, ZPT)], accum.at[pl.ds(r0, ZPT)])
    pltpu.sync_copy(src_hbm.at[w], si_v)
    pltpu.sync_copy(dst_hbm.at[w], di_v)
    plsc.subcore_barrier()

    def step(k, _):
        pltpu.async_copy(g_hbm.at[si_v.at[k]], rows_v, sem).wait()
        pltpu.sync_copy(rows_v, accum.at[di_v.at[k]], add=True)
        return 0

    lax.fori_loop(0, RPW, step, 0)
    plsc.subcore_barrier()
    pltpu.sync_copy(accum.at[pl.ds(r0, ZPT)], out_hbm.at[c, pl.ds(r0, ZPT)])


_BLK = 2000
_GRID = N // _BLK


_DEG_SPEC = lambda i: (0, i, 0)


def _dinv_block(deg_ref):
    d = deg_ref[0][:, 0:1] + deg_ref[1][:, 0:1] + 1.0
    return 1.0 / jnp.sqrt(d)


def _g_body(deg_ref, x_ref, g_ref):
    g_ref[...] = x_ref[...] * _dinv_block(deg_ref)


def _g_call(deg2, x):
    return pl.pallas_call(
        _g_body,
        grid=(_GRID,),
        in_specs=[
            pl.BlockSpec((NC, _BLK, IN_C), lambda i: (0, i, 0)),
            pl.BlockSpec((_BLK, IN_C), lambda i: (i, 0)),
        ],
        out_specs=pl.BlockSpec((_BLK, IN_C), lambda i: (i, 0)),
        out_shape=jax.ShapeDtypeStruct((N, IN_C), jnp.float32),
    )(deg2, x)


def _stats_body(deg_ref, agg_ref, g_ref, W1_ref, b1_ref, gm_ref, bt_ref,
                a_ref, c_ref, s1, s2):
    i = pl.program_id(0)
    dinv = _dinv_block(deg_ref)
    u = (agg_ref[0] + agg_ref[1] + g_ref[...]) * dinv

    @pl.when(i == 0)
    def _():
        s1[...] = jnp.zeros_like(s1)
        s2[...] = jnp.zeros_like(s2)

    s1[...] += jnp.sum(u, axis=0, keepdims=True)
    s2[...] += lax.dot_general(u, u, (((0,), (0,)), ((), ())),
                               preferred_element_type=jnp.float32, precision=jax.lax.Precision.HIGHEST)

    @pl.when(i == pl.num_programs(0) - 1)
    def _():
        W1 = W1_ref[...]
        b1 = b1_ref[...]
        m = s1[...] * (1.0 / N)
        mean_h = lax.dot_general(m, W1, (((1,), (1,)), ((), ())),
                                 preferred_element_type=jnp.float32, precision=jax.lax.Precision.HIGHEST) + b1
        T = jnp.dot(W1, s2[...] * (1.0 / N), preferred_element_type=jnp.float32, precision=jax.lax.Precision.HIGHEST)
        q = jnp.sum(T * W1, axis=1)[None, :]
        var = q + 2.0 * b1 * mean_h - b1 * b1 - mean_h * mean_h
        a = gm_ref[...] / jnp.sqrt(var + 1e-5)
        a_ref[...] = a
        c_ref[...] = bt_ref[...] - mean_h * a + a * b1


def _stats_call(deg2, aggp, g, W1, b1, gamma, beta):
    return pl.pallas_call(
        _stats_body,
        grid=(_GRID,),
        in_specs=[
            pl.BlockSpec((NC, _BLK, IN_C), lambda i: (0, i, 0)),
            pl.BlockSpec((NC, _BLK, IN_C), lambda i: (0, i, 0)),
            pl.BlockSpec((_BLK, IN_C), lambda i: (i, 0)),
            pl.BlockSpec((HID, IN_C), lambda i: (0, 0)),
            pl.BlockSpec((1, HID), lambda i: (0, 0)),
            pl.BlockSpec((1, HID), lambda i: (0, 0)),
            pl.BlockSpec((1, HID), lambda i: (0, 0)),
        ],
        out_specs=[
            pl.BlockSpec((1, HID), lambda i: (0, 0)),
            pl.BlockSpec((1, HID), lambda i: (0, 0)),
        ],
        out_shape=[
            jax.ShapeDtypeStruct((1, HID), jnp.float32),
            jax.ShapeDtypeStruct((1, HID), jnp.float32),
        ],
        scratch_shapes=[
            pltpu.VMEM((1, IN_C), jnp.float32),
            pltpu.VMEM((IN_C, IN_C), jnp.float32),
        ],
        compiler_params=pltpu.CompilerParams(
            dimension_semantics=("arbitrary",)),
    )(deg2, aggp, g, W1, b1, gamma, beta)


def _out_body(deg_ref, agg_ref, g_ref, W1t_ref, a_ref, c_ref, W2t_ref, b2_ref,
              o_ref):
    dinv = _dinv_block(deg_ref)
    u = (agg_ref[0] + agg_ref[1] + g_ref[...]) * dinv
    h = jnp.dot(u, W1t_ref[...], preferred_element_type=jnp.float32, precision=jax.lax.Precision.HIGHEST)
    h = jnp.maximum(h * a_ref[...] + c_ref[...], 0.0)
    o_ref[...] = (jnp.dot(h, W2t_ref[...], preferred_element_type=jnp.float32, precision=jax.lax.Precision.HIGHEST)
                  + b2_ref[...])


def _out_call(deg2, aggp, g, W1t, a, cf, W2t, b2):
    return pl.pallas_call(
        _out_body,
        grid=(_GRID,),
        in_specs=[
            pl.BlockSpec((NC, _BLK, IN_C), lambda i: (0, i, 0)),
            pl.BlockSpec((NC, _BLK, IN_C), lambda i: (0, i, 0)),
            pl.BlockSpec((_BLK, IN_C), lambda i: (i, 0)),
            pl.BlockSpec((IN_C, HID), lambda i: (0, 0)),
            pl.BlockSpec((1, HID), lambda i: (0, 0)),
            pl.BlockSpec((1, HID), lambda i: (0, 0)),
            pl.BlockSpec((HID, OUT_C), lambda i: (0, 0)),
            pl.BlockSpec((1, OUT_C), lambda i: (0, 0)),
        ],
        out_specs=pl.BlockSpec((_BLK, OUT_C), lambda i: (i, 0)),
        out_shape=jax.ShapeDtypeStruct((N, OUT_C), jnp.float32),
    )(deg2, aggp, g, W1t, a, cf, W2t, b2)


def kernel(x, edge_index, W1, b1, gamma, beta, W2, b2):
    src = edge_index[0]
    dst = edge_index[1]
    pad = E_PAD - E
    ar = jnp.arange(pad, dtype=jnp.int32)
    src_p = jnp.concatenate([src, ar % jnp.int32(N)])
    dst_p = jnp.concatenate([dst, jnp.int32(N) + (ar % jnp.int32(NP2 - N))])
    src3 = src_p.reshape(NW, RPW, CH)
    dst3 = dst_p.reshape(NW, RPW, CH)
    z128 = jnp.zeros((NP2, IN_C), jnp.float32)
    ones = jnp.ones((CH, IN_C), jnp.float32)

    deg2 = _deg_sc(dst3, ones, z128)
    g = _g_call(deg2, x)
    aggp = _agg_sc(src3, dst3, g, z128)
    a, cf = _stats_call(deg2, aggp, g, W1,
                        b1.reshape(1, HID), gamma.reshape(1, HID),
                        beta.reshape(1, HID))
    return _out_call(deg2, aggp, g, W1.T, a, cf, W2.T, b2.reshape(1, OUT_C))


# speedup vs baseline: 1.1389x; 1.1389x over previous
import functools

import jax
import jax.numpy as jnp
from jax import lax
from jax.experimental import pallas as pl
from jax.experimental.pallas import tpu as pltpu
from jax.experimental.pallas import tpu_sc as plsc

N = 10000
E = 320000
IN_C = 128
HID = 256
OUT_C = 64

NC = 2
NS = 16
NW = NC * NS
CH = 128
NG = 8
G = 10
RPW = NG * G
E_PAD = NW * RPW * CH
NP2 = 10240
ZPT = NP2 // NS

_MESH = plsc.VectorSubcoreMesh(core_axis_name="c", subcore_axis_name="s")


@functools.partial(
    pl.kernel,
    mesh=_MESH,
    out_type=jax.ShapeDtypeStruct((NC, NP2, IN_C), jnp.float32),
    scratch_types=[
        pltpu.VMEM((RPW, CH), jnp.int32),
        pltpu.VMEM((CH, IN_C), jnp.float32),
        pltpu.VMEM_SHARED((NP2, IN_C), jnp.float32),
    ],
)
def _deg_sc(dst_hbm, ones_hbm, z_hbm, out_hbm, idx_v, ones_v, accum):
    c = lax.axis_index("c")
    s = lax.axis_index("s")
    w = c * NS + s
    r0 = s * ZPT
    pltpu.sync_copy(z_hbm.at[pl.ds(r0, ZPT)], accum.at[pl.ds(r0, ZPT)])
    pltpu.sync_copy(dst_hbm.at[w], idx_v)
    pltpu.sync_copy(ones_hbm, ones_v)
    plsc.subcore_barrier()

    def step(k, _):
        pltpu.sync_copy(ones_v, accum.at[idx_v.at[k]], add=True)
        return 0

    lax.fori_loop(0, RPW, step, 0)
    plsc.subcore_barrier()
    pltpu.sync_copy(accum.at[pl.ds(r0, ZPT)], out_hbm.at[c, pl.ds(r0, ZPT)])


@functools.partial(
    pl.kernel,
    mesh=_MESH,
    out_type=jax.ShapeDtypeStruct((NC, NP2, IN_C), jnp.float32),
    scratch_types=[
        pltpu.VMEM((2, 2, G, CH), jnp.int32),
        pltpu.VMEM((CH, IN_C), jnp.float32),
        pltpu.VMEM((CH, IN_C), jnp.float32),
        pltpu.VMEM_SHARED((NP2, IN_C), jnp.float32),
        pltpu.SemaphoreType.DMA,
        pltpu.SemaphoreType.DMA,
        pltpu.SemaphoreType.DMA,
    ],
)
def _agg_sc(ei_hbm, g_hbm, z_hbm, out_hbm, ei_v, rows_a, rows_b, accum,
            sem_a, sem_b, sem_i):
    c = lax.axis_index("c")
    s = lax.axis_index("s")
    w = c * NS + s
    r0 = s * ZPT
    pltpu.sync_copy(z_hbm.at[pl.ds(r0, ZPT)], accum.at[pl.ds(r0, ZPT)])

    def stage(q, slot):
        return pltpu.make_async_copy(ei_hbm.at[w, q], ei_v.at[slot], sem_i)

    def gather(slot, l, buf, sem):
        return pltpu.make_async_copy(g_hbm.at[ei_v.at[slot, 0, l]], buf, sem)

    def scatter(slot, l, buf):
        pltpu.sync_copy(buf, accum.at[ei_v.at[slot, 1, l]], add=True)

    stage(0, 0).start()
    stage(0, 0).wait()
    stage(1, 1).start()
    plsc.subcore_barrier()
    gather(0, 0, rows_a, sem_a).start()

    for q in range(NG):
        cur, oth = q % 2, (q + 1) % 2

        def inner(p, _, cur=cur, oth=oth, q=q):
            l0 = 2 * p
            gather(cur, l0, rows_a, sem_a).wait()
            gather(cur, l0 + 1, rows_b, sem_b).start()
            scatter(cur, l0, rows_a)
            gather(cur, l0 + 1, rows_b, sem_b).wait()

            @pl.when(p < G // 2 - 1)
            def _():
                gather(cur, l0 + 2, rows_a, sem_a).start()

            @pl.when(p == G // 2 - 1)
            def _():
                if q == NG - 1:
                    gather(cur, G - 1, rows_a, sem_a).start()
                else:
                    stage(q + 1, oth).wait()
                    gather(oth, 0, rows_a, sem_a).start()

            scatter(cur, l0 + 1, rows_b)
            return 0

        lax.fori_loop(0, G // 2, inner, 0)
        if q < NG - 2:
            stage(q + 2, cur).start()

    gather(0, G - 1, rows_a, sem_a).wait()
    plsc.subcore_barrier()
    pltpu.sync_copy(accum.at[pl.ds(r0, ZPT)], out_hbm.at[c, pl.ds(r0, ZPT)])


_BLK = 2000
_GRID = N // _BLK


_DEG_SPEC = lambda i: (0, i, 0)


def _dinv_block(deg_ref):
    d = deg_ref[0][:, 0:1] + deg_ref[1][:, 0:1] + 1.0
    return 1.0 / jnp.sqrt(d)


def _g_body(deg_ref, x_ref, g_ref):
    g_ref[...] = x_ref[...] * _dinv_block(deg_ref)


def _g_call(deg2, x):
    return pl.pallas_call(
        _g_body,
        grid=(_GRID,),
        in_specs=[
            pl.BlockSpec((NC, _BLK, IN_C), lambda i: (0, i, 0)),
            pl.BlockSpec((_BLK, IN_C), lambda i: (i, 0)),
        ],
        out_specs=pl.BlockSpec((_BLK, IN_C), lambda i: (i, 0)),
        out_shape=jax.ShapeDtypeStruct((N, IN_C), jnp.float32),
    )(deg2, x)


def _stats_body(deg_ref, agg_ref, g_ref, W1_ref, b1_ref, gm_ref, bt_ref,
                a_ref, c_ref, s1, s2):
    i = pl.program_id(0)
    dinv = _dinv_block(deg_ref)
    u = (agg_ref[0] + agg_ref[1] + g_ref[...]) * dinv

    @pl.when(i == 0)
    def _():
        s1[...] = jnp.zeros_like(s1)
        s2[...] = jnp.zeros_like(s2)

    s1[...] += jnp.sum(u, axis=0, keepdims=True)
    s2[...] += lax.dot_general(u, u, (((0,), (0,)), ((), ())),
                               preferred_element_type=jnp.float32, precision=jax.lax.Precision.HIGHEST)

    @pl.when(i == pl.num_programs(0) - 1)
    def _():
        W1 = W1_ref[...]
        b1 = b1_ref[...]
        m = s1[...] * (1.0 / N)
        mean_h = lax.dot_general(m, W1, (((1,), (1,)), ((), ())),
                                 preferred_element_type=jnp.float32, precision=jax.lax.Precision.HIGHEST) + b1
        T = jnp.dot(W1, s2[...] * (1.0 / N), preferred_element_type=jnp.float32, precision=jax.lax.Precision.HIGHEST)
        q = jnp.sum(T * W1, axis=1)[None, :]
        var = q + 2.0 * b1 * mean_h - b1 * b1 - mean_h * mean_h
        a = gm_ref[...] / jnp.sqrt(var + 1e-5)
        a_ref[...] = a
        c_ref[...] = bt_ref[...] - mean_h * a + a * b1


def _stats_call(deg2, aggp, g, W1, b1, gamma, beta):
    return pl.pallas_call(
        _stats_body,
        grid=(_GRID,),
        in_specs=[
            pl.BlockSpec((NC, _BLK, IN_C), lambda i: (0, i, 0)),
            pl.BlockSpec((NC, _BLK, IN_C), lambda i: (0, i, 0)),
            pl.BlockSpec((_BLK, IN_C), lambda i: (i, 0)),
            pl.BlockSpec((HID, IN_C), lambda i: (0, 0)),
            pl.BlockSpec((1, HID), lambda i: (0, 0)),
            pl.BlockSpec((1, HID), lambda i: (0, 0)),
            pl.BlockSpec((1, HID), lambda i: (0, 0)),
        ],
        out_specs=[
            pl.BlockSpec((1, HID), lambda i: (0, 0)),
            pl.BlockSpec((1, HID), lambda i: (0, 0)),
        ],
        out_shape=[
            jax.ShapeDtypeStruct((1, HID), jnp.float32),
            jax.ShapeDtypeStruct((1, HID), jnp.float32),
        ],
        scratch_shapes=[
            pltpu.VMEM((1, IN_C), jnp.float32),
            pltpu.VMEM((IN_C, IN_C), jnp.float32),
        ],
        compiler_params=pltpu.CompilerParams(
            dimension_semantics=("arbitrary",)),
    )(deg2, aggp, g, W1, b1, gamma, beta)


def _out_body(deg_ref, agg_ref, g_ref, W1t_ref, a_ref, c_ref, W2t_ref, b2_ref,
              o_ref):
    dinv = _dinv_block(deg_ref)
    u = (agg_ref[0] + agg_ref[1] + g_ref[...]) * dinv
    h = jnp.dot(u, W1t_ref[...], preferred_element_type=jnp.float32, precision=jax.lax.Precision.HIGHEST)
    h = jnp.maximum(h * a_ref[...] + c_ref[...], 0.0)
    o_ref[...] = (jnp.dot(h, W2t_ref[...], preferred_element_type=jnp.float32, precision=jax.lax.Precision.HIGHEST)
                  + b2_ref[...])


def _out_call(deg2, aggp, g, W1t, a, cf, W2t, b2):
    return pl.pallas_call(
        _out_body,
        grid=(_GRID,),
        in_specs=[
            pl.BlockSpec((NC, _BLK, IN_C), lambda i: (0, i, 0)),
            pl.BlockSpec((NC, _BLK, IN_C), lambda i: (0, i, 0)),
            pl.BlockSpec((_BLK, IN_C), lambda i: (i, 0)),
            pl.BlockSpec((IN_C, HID), lambda i: (0, 0)),
            pl.BlockSpec((1, HID), lambda i: (0, 0)),
            pl.BlockSpec((1, HID), lambda i: (0, 0)),
            pl.BlockSpec((HID, OUT_C), lambda i: (0, 0)),
            pl.BlockSpec((1, OUT_C), lambda i: (0, 0)),
        ],
        out_specs=pl.BlockSpec((_BLK, OUT_C), lambda i: (i, 0)),
        out_shape=jax.ShapeDtypeStruct((N, OUT_C), jnp.float32),
    )(deg2, aggp, g, W1t, a, cf, W2t, b2)


def kernel(x, edge_index, W1, b1, gamma, beta, W2, b2):
    src = edge_index[0]
    dst = edge_index[1]
    pad = E_PAD - E
    ar = jnp.arange(pad, dtype=jnp.int32)
    src_p = jnp.concatenate([src, ar % jnp.int32(N)])
    dst_p = jnp.concatenate([dst, jnp.int32(N) + (ar % jnp.int32(NP2 - N))])
    dst3 = dst_p.reshape(NW, RPW, CH)
    ei4 = jnp.stack([src_p.reshape(NW, NG, G, CH),
                     dst_p.reshape(NW, NG, G, CH)], axis=2)
    z128 = jnp.zeros((NP2, IN_C), jnp.float32)
    ones = jnp.ones((CH, IN_C), jnp.float32)

    deg2 = _deg_sc(dst3, ones, z128)
    g = _g_call(deg2, x)
    aggp = _agg_sc(ei4, g, z128)
    a, cf = _stats_call(deg2, aggp, g, W1,
                        b1.reshape(1, HID), gamma.reshape(1, HID),
                        beta.reshape(1, HID))
    return _out_call(deg2, aggp, g, W1.T, a, cf, W2.T, b2.reshape(1, OUT_C))
